# Initial kernel scaffold; baseline (speedup 1.0000x reference)
import jax
import jax.numpy as jnp
from jax.experimental import pallas as pl


def kernel(x, edge_index, weight):
    raise NotImplementedError("write your pallas kernel here")


# speedup vs baseline: 4.6032x; 4.6032x over previous
import functools

import jax
import jax.numpy as jnp
from jax import lax
from jax.experimental import pallas as pl
from jax.experimental.pallas import tpu as pltpu
from jax.experimental.pallas import tpu_sc as plsc

N = 10000
E = 320000
D = 128
DP = 144
L = 16

NC = 2
NS = 16
NW = NC * NS

CH = 80
EDGES_PER_W = E // NW
CHUNKS = EDGES_PER_W // CH
NP_ = 10240
ROWS_PER_TILE = NP_ // NS


def _bcast_lane(v, e):
    return lax.gather(
        v,
        jnp.full((L, 1), e, dtype=jnp.int32),
        lax.GatherDimensionNumbers(
            offset_dims=(), collapsed_slice_dims=(0,), start_index_map=(0,)),
        (1,),
        mode=lax.GatherScatterMode.PROMISE_IN_BOUNDS,
    )


def _phase1_body(row_hbm, col_hbm, w_hbm, xpad_hbm, zeros_hbm, acc_hbm,
                 cidx_v, ridx_v, w_v, rows_v, acc_sh, sem):
    cid = lax.axis_index("c")
    sid = lax.axis_index("s")
    wid = cid * NS + sid

    r0 = sid * ROWS_PER_TILE
    pltpu.sync_copy(zeros_hbm.at[pl.ds(r0, ROWS_PER_TILE)],
                    acc_sh.at[pl.ds(r0, ROWS_PER_TILE)])
    plsc.subcore_barrier()

    base_e = wid * EDGES_PER_W

    def chunk_body(i, _):
        off = base_e + i * CH
        pltpu.sync_copy(col_hbm.at[pl.ds(off, CH)], cidx_v)
        pltpu.sync_copy(row_hbm.at[pl.ds(off, CH)], ridx_v)
        pltpu.sync_copy(w_hbm.at[pl.ds(off, CH)], w_v)
        pltpu.async_copy(xpad_hbm.at[cidx_v], rows_v, sem).wait()
        for k in range(CH // L):
            vals = jnp.exp(w_v[pl.ds(k * L, L)])
            for e in range(L):
                b = _bcast_lane(vals, e)
                row = k * L + e
                for j in range(DP // L):
                    rows_v[row, pl.ds(j * L, L)] = (
                        rows_v[row, pl.ds(j * L, L)] * b)
        pltpu.sync_copy(rows_v, acc_sh.at[ridx_v], add=True)
        return _

    lax.fori_loop(0, CHUNKS, chunk_body, 0)
    plsc.subcore_barrier()

    pltpu.sync_copy(acc_sh.at[pl.ds(r0, ROWS_PER_TILE)],
                    acc_hbm.at[cid, pl.ds(r0, ROWS_PER_TILE)])


def _norm_body(acc_ref, out_ref):
    a = acc_ref[0]
    b = acc_ref[1]
    s = a[:, 128:129] + b[:, 128:129] + 1e-20
    out_ref[...] = (a[:, :D] + b[:, :D]) / s


def kernel(x, edge_index, weight):
    xp = jnp.concatenate(
        [x[0],
         jnp.ones((N, 1), jnp.float32),
         jnp.zeros((N, DP - D - 1), jnp.float32)], axis=1)
    row = edge_index[0]
    col = edge_index[1]
    zeros = jnp.zeros((NP_, DP), jnp.float32)

    mesh = plsc.VectorSubcoreMesh(core_axis_name="c", subcore_axis_name="s")
    phase1 = functools.partial(
        pl.kernel, mesh=mesh,
        compiler_params=pltpu.CompilerParams(use_tc_tiling_on_sc=False),
        out_type=jax.ShapeDtypeStruct((NC, NP_, DP), jnp.float32),
        scratch_types=[
            pltpu.VMEM((CH,), jnp.int32),
            pltpu.VMEM((CH,), jnp.int32),
            pltpu.VMEM((CH,), jnp.float32),
            pltpu.VMEM((CH, DP), jnp.float32),
            pltpu.VMEM_SHARED((NP_, DP), jnp.float32),
            pltpu.SemaphoreType.DMA,
        ],
    )(_phase1_body)
    acc = phase1(row, col, weight, xp, zeros)

    bn = 400
    out = pl.pallas_call(
        _norm_body,
        grid=(N // bn,),
        in_specs=[pl.BlockSpec((NC, bn, DP), lambda i: (0, i, 0))],
        out_specs=pl.BlockSpec((bn, D), lambda i: (i, 0)),
        out_shape=jax.ShapeDtypeStruct((N, D), jnp.float32),
    )(acc)
    return out[None]
